# Initial kernel scaffold; baseline (speedup 1.0000x reference)
import jax
import jax.numpy as jnp
from jax.experimental import pallas as pl


def kernel(loc, conf, priors):
    raise NotImplementedError("write your pallas kernel here")


# speedup vs baseline: 16.3068x; 16.3068x over previous
import functools

import jax
import jax.numpy as jnp
from jax.experimental import pallas as pl
from jax.experimental.pallas import tpu as pltpu

_CONF_THRESH = 0.5
_IOU_THRESH = 0.3
_VAR0, _VAR1 = 0.1, 0.2
_MAX_DET = 100
_LANES = 128


def _nms_kernel(n_valid, l0, l1, l2, l3, c1, p0, p1, p2, p3,
                out_ref, s_ref, x1_ref, y1_ref, x2_ref, y2_ref, ar_ref):
    rows = l0.shape[0]

    p2v = p2[...]
    p3v = p3[...]
    cx = p0[...] + l0[...] * _VAR0 * p2v
    cy = p1[...] + l1[...] * _VAR0 * p3v
    w = p2v * jnp.exp(l2[...] * _VAR1)
    h = p3v * jnp.exp(l3[...] * _VAR1)
    x1 = cx - w / 2.0
    y1 = cy - h / 2.0
    x2 = cx + w / 2.0
    y2 = cy + h / 2.0

    row_i = jax.lax.broadcasted_iota(jnp.int32, (rows, _LANES), 0)
    col_i = jax.lax.broadcasted_iota(jnp.int32, (rows, _LANES), 1)
    lin = row_i * _LANES + col_i
    in_bounds = lin < n_valid

    probs = jax.nn.sigmoid(c1[...])
    score = jnp.where(probs >= _CONF_THRESH, probs, 0.0)
    score = jnp.where(in_bounds, score, 0.0)

    area = jnp.maximum(x2 - x1, 0.0) * jnp.maximum(y2 - y1, 0.0)

    x1_ref[...] = x1
    y1_ref[...] = y1
    x2_ref[...] = x2
    y2_ref[...] = y2
    ar_ref[...] = area
    s_ref[...] = score

    big = jnp.int32(rows * _LANES + 1)

    def step(i, _):
        s = s_ref[...]
        m = jnp.max(s)
        idx = jnp.min(jnp.where(s == m, lin, big))
        sel = lin == idx

        x1v = x1_ref[...]
        y1v = y1_ref[...]
        x2v = x2_ref[...]
        y2v = y2_ref[...]
        arv = ar_ref[...]

        zero = jnp.float32(0.0)
        bx1 = jnp.sum(jnp.where(sel, x1v, zero))
        by1 = jnp.sum(jnp.where(sel, y1v, zero))
        bx2 = jnp.sum(jnp.where(sel, x2v, zero))
        by2 = jnp.sum(jnp.where(sel, y2v, zero))
        bar = jnp.sum(jnp.where(sel, arv, zero))

        xx1 = jnp.maximum(bx1, x1v)
        yy1 = jnp.maximum(by1, y1v)
        xx2 = jnp.minimum(bx2, x2v)
        yy2 = jnp.minimum(by2, y2v)
        iw = jnp.maximum(xx2 - xx1, 0.0)
        ih = jnp.maximum(yy2 - yy1, 0.0)
        inter = iw * ih
        iou = inter / (bar + arv - inter + 1e-9)
        supp = jnp.logical_or(iou > _IOU_THRESH, sel)
        s_ref[...] = jnp.where(supp, -1.0, s)

        valid = m > 0.0
        li = jax.lax.broadcasted_iota(jnp.int32, (1, _LANES), 1)
        rowvec = (jnp.where(li == 0, bx1, zero)
                  + jnp.where(li == 1, by1, zero)
                  + jnp.where(li == 2, bx2, zero)
                  + jnp.where(li == 3, by2, zero)
                  + jnp.where(li == 4, m, zero))
        out_ref[pl.ds(i, 1), :] = jnp.where(valid, rowvec, zero)
        return 0

    jax.lax.fori_loop(0, _MAX_DET, step, 0)


@jax.jit
def kernel(loc, conf, priors):
    n = loc.shape[0]
    rows = (n + _LANES - 1) // _LANES
    rows = ((rows + 7) // 8) * 8
    n_pad = rows * _LANES

    def col(a, j, fill):
        c = a[:, j]
        c = jnp.concatenate([c, jnp.full((n_pad - n,), fill, c.dtype)])
        return c.reshape(rows, _LANES)

    args = (
        col(loc, 0, 0.0), col(loc, 1, 0.0), col(loc, 2, 0.0), col(loc, 3, 0.0),
        col(conf, 1, -1e9),
        col(priors, 0, 0.0), col(priors, 1, 0.0), col(priors, 2, 0.0), col(priors, 3, 0.0),
    )

    scratch = [pltpu.VMEM((rows, _LANES), jnp.float32)] * 6
    out = pl.pallas_call(
        functools.partial(_nms_kernel, n),
        out_shape=jax.ShapeDtypeStruct((_MAX_DET, _LANES), jnp.float32),
        scratch_shapes=scratch,
    )(*args)
    return out[:, :5]


# speedup vs baseline: 17.2356x; 1.0570x over previous
import functools

import jax
import jax.numpy as jnp
from jax.experimental import pallas as pl
from jax.experimental.pallas import tpu as pltpu

_CONF_THRESH = 0.5
_IOU_THRESH = 0.3
_VAR0, _VAR1 = 0.1, 0.2
_MAX_DET = 100
_LANES = 128


def _nms_kernel(n_valid, l0, l1, l2, l3, c1, p0, p1, p2, p3,
                out_ref, s_ref, x1_ref, y1_ref, x2_ref, y2_ref, ar_ref):
    rows = l0.shape[0]

    p2v = p2[...]
    p3v = p3[...]
    cx = p0[...] + l0[...] * _VAR0 * p2v
    cy = p1[...] + l1[...] * _VAR0 * p3v
    w = p2v * jnp.exp(l2[...] * _VAR1)
    h = p3v * jnp.exp(l3[...] * _VAR1)
    x1 = cx - w / 2.0
    y1 = cy - h / 2.0
    x2 = cx + w / 2.0
    y2 = cy + h / 2.0

    row_i = jax.lax.broadcasted_iota(jnp.int32, (rows, _LANES), 0)
    col_i = jax.lax.broadcasted_iota(jnp.int32, (rows, _LANES), 1)
    lin = row_i * _LANES + col_i
    in_bounds = lin < n_valid

    probs = jax.nn.sigmoid(c1[...])
    score = jnp.where(probs >= _CONF_THRESH, probs, 0.0)
    score = jnp.where(in_bounds, score, 0.0)

    area = jnp.maximum(x2 - x1, 0.0) * jnp.maximum(y2 - y1, 0.0)

    x1_ref[...] = x1
    y1_ref[...] = y1
    x2_ref[...] = x2
    y2_ref[...] = y2
    ar_ref[...] = area
    s_ref[...] = score

    big = jnp.int32(rows * _LANES + 1)
    col1 = jax.lax.broadcasted_iota(jnp.int32, (1, _LANES), 1)

    def step(i, m):
        s = s_ref[...]
        idx = jnp.min(jnp.where(s == m, lin, big))
        sel = lin == idx

        x1v = x1_ref[...]
        y1v = y1_ref[...]
        x2v = x2_ref[...]
        y2v = y2_ref[...]
        arv = ar_ref[...]

        zero = jnp.float32(0.0)
        r = idx // _LANES
        selc = col1 == (idx - r * _LANES)
        bx1 = jnp.sum(jnp.where(selc, x1_ref[pl.ds(r, 1), :], zero))
        by1 = jnp.sum(jnp.where(selc, y1_ref[pl.ds(r, 1), :], zero))
        bx2 = jnp.sum(jnp.where(selc, x2_ref[pl.ds(r, 1), :], zero))
        by2 = jnp.sum(jnp.where(selc, y2_ref[pl.ds(r, 1), :], zero))
        bar = jnp.sum(jnp.where(selc, ar_ref[pl.ds(r, 1), :], zero))

        xx1 = jnp.maximum(bx1, x1v)
        yy1 = jnp.maximum(by1, y1v)
        xx2 = jnp.minimum(bx2, x2v)
        yy2 = jnp.minimum(by2, y2v)
        iw = jnp.maximum(xx2 - xx1, 0.0)
        ih = jnp.maximum(yy2 - yy1, 0.0)
        inter = iw * ih
        iou = inter / (bar + arv - inter + 1e-9)
        supp = jnp.logical_or(iou > _IOU_THRESH, sel)
        s_new = jnp.where(supp, -1.0, s)
        s_ref[...] = s_new

        valid = m > 0.0
        li = jax.lax.broadcasted_iota(jnp.int32, (1, _LANES), 1)
        rowvec = (jnp.where(li == 0, bx1, zero)
                  + jnp.where(li == 1, by1, zero)
                  + jnp.where(li == 2, bx2, zero)
                  + jnp.where(li == 3, by2, zero)
                  + jnp.where(li == 4, m, zero))
        out_ref[pl.ds(i, 1), :] = jnp.where(valid, rowvec, zero)
        return jnp.max(s_new)

    jax.lax.fori_loop(0, _MAX_DET, step, jnp.max(score))


@jax.jit
def kernel(loc, conf, priors):
    n = loc.shape[0]
    rows = (n + _LANES - 1) // _LANES
    rows = ((rows + 7) // 8) * 8
    n_pad = rows * _LANES

    def col(a, j, fill):
        c = a[:, j]
        c = jnp.concatenate([c, jnp.full((n_pad - n,), fill, c.dtype)])
        return c.reshape(rows, _LANES)

    args = (
        col(loc, 0, 0.0), col(loc, 1, 0.0), col(loc, 2, 0.0), col(loc, 3, 0.0),
        col(conf, 1, -1e9),
        col(priors, 0, 0.0), col(priors, 1, 0.0), col(priors, 2, 0.0), col(priors, 3, 0.0),
    )

    scratch = [pltpu.VMEM((rows, _LANES), jnp.float32)] * 6
    out = pl.pallas_call(
        functools.partial(_nms_kernel, n),
        out_shape=jax.ShapeDtypeStruct((_MAX_DET, _LANES), jnp.float32),
        scratch_shapes=scratch,
    )(*args)
    return out[:, :5]
